# baseline (speedup 1.0000x reference)
import jax
import jax.numpy as jnp
import numpy as np
from jax import lax
from jax.experimental import pallas as pl
from jax.experimental.pallas import tpu as pltpu
from jax.experimental.pallas import tpu_sc as plsc
from jax._src.random.threefry2x32 import threefry2x32_p

KC = 64
LANES = 16
NCORES = 2
NSUB = 16
NW = NCORES * NSUB
BB = 4
NN = 65536
CPB = NW // BB
CH = NN // CPB
GROUPS = CH // LANES
ACC = KC * LANES
EW = 2 * KC


def _reduce_lanes(ref, colbase, off):
    def body(c, acc):
        return acc + plsc.load_gather(ref, [colbase + (off + c)])
    return lax.fori_loop(1, LANES, body, plsc.load_gather(ref, [colbase + off]))


def _combine_partials(s, b, part_v, tmp_v, shp, out_hbm, nvec):
    width = nvec * LANES
    pltpu.sync_copy(part_v, shp.at[pl.ds(s * width, width)])
    plsc.subcore_barrier()

    @pl.when(s % CPB == 0)
    def _():
        pltpu.sync_copy(shp.at[pl.ds(s * width, CPB * width)], tmp_v)

        def vbody(v, carry):
            def jbody(j, acc):
                return acc + tmp_v[pl.ds(j * width + v * LANES, LANES)]
            part_v[pl.ds(v * LANES, LANES)] = lax.fori_loop(
                1, CPB, jbody, tmp_v[pl.ds(v * LANES, LANES)])
            return carry

        lax.fori_loop(0, nvec, vbody, 0)
        pltpu.sync_copy(part_v.at[pl.ds(0, width)], out_hbm.at[b])


def _counts_body(zs_hbm, out_hbm, zs_v, cnt_v, part_v, tmp_v, shp):
    s = lax.axis_index("s")
    wid = lax.axis_index("c") * NSUB + s
    b = wid // CPB
    start = (wid % CPB) * CH

    pltpu.sync_copy(zs_hbm.at[b, pl.ds(start, CH)], zs_v)

    lane = lax.iota(jnp.int32, LANES)
    colbase = lane * LANES
    zeros = jnp.zeros((LANES,), jnp.float32)
    ones = jnp.ones((LANES,), jnp.float32)

    @plsc.parallel_loop(0, ACC // LANES, unroll=4)
    def _(i):
        cnt_v[pl.ds(i * LANES, LANES)] = zeros

    @plsc.parallel_loop(0, GROUPS, unroll=4)
    def _(i):
        z = zs_v[pl.ds(i * LANES, LANES)]
        plsc.addupdate_scatter(cnt_v, [z * LANES + lane], ones)

    for ch in range(KC // LANES):
        v = _reduce_lanes(cnt_v, colbase, ch * LANES * LANES)
        idx = lane * 2 + (ch * 2 * LANES)
        plsc.store_scatter(part_v, [idx], v)
        plsc.store_scatter(part_v, [idx + 1], v)

    _combine_partials(s, b, part_v, tmp_v, shp, out_hbm, EW // LANES)


def _sums_body(zs_hbm, x0_hbm, x1_hbm, out_hbm,
               zs_v, x0_v, x1_v, sx0_v, sx1_v, sq0_v, sq1_v, part_v, tmp_v, shp):
    s = lax.axis_index("s")
    wid = lax.axis_index("c") * NSUB + s
    b = wid // CPB
    start = (wid % CPB) * CH

    pltpu.sync_copy(zs_hbm.at[b, pl.ds(start, CH)], zs_v)
    pltpu.sync_copy(x0_hbm.at[b, pl.ds(start, CH)], x0_v)
    pltpu.sync_copy(x1_hbm.at[b, pl.ds(start, CH)], x1_v)

    lane = lax.iota(jnp.int32, LANES)
    colbase = lane * LANES
    zeros = jnp.zeros((LANES,), jnp.float32)

    @plsc.parallel_loop(0, ACC // LANES, unroll=4)
    def _(i):
        sl = pl.ds(i * LANES, LANES)
        sx0_v[sl] = zeros
        sx1_v[sl] = zeros
        sq0_v[sl] = zeros
        sq1_v[sl] = zeros

    @plsc.parallel_loop(0, GROUPS, unroll=2)
    def _(i):
        sl = pl.ds(i * LANES, LANES)
        z = zs_v[sl]
        x0 = x0_v[sl]
        x1 = x1_v[sl]
        idx = z * LANES + lane
        plsc.addupdate_scatter(sx0_v, [idx], x0)
        plsc.addupdate_scatter(sx1_v, [idx], x1)
        plsc.addupdate_scatter(sq0_v, [idx], x0 * x0)
        plsc.addupdate_scatter(sq1_v, [idx], x1 * x1)

    for ch in range(KC // LANES):
        off = ch * LANES * LANES
        base = ch * 2 * LANES
        idx = lane * 2 + base
        plsc.store_scatter(part_v, [idx], _reduce_lanes(sx0_v, colbase, off))
        plsc.store_scatter(part_v, [idx + 1], _reduce_lanes(sx1_v, colbase, off))
        plsc.store_scatter(part_v, [idx + EW], _reduce_lanes(sq0_v, colbase, off))
        plsc.store_scatter(part_v, [idx + EW + 1], _reduce_lanes(sq1_v, colbase, off))

    _combine_partials(s, b, part_v, tmp_v, shp, out_hbm, 2 * EW // LANES)


_F3 = np.float32(1.0 / 3.0)
_SQUEEZE = np.float32(0.0331)
_NLO = np.nextafter(np.float32(-1.0), np.float32(0.0), dtype=np.float32)
_SQRT2 = np.float32(np.sqrt(2.0))


def _tf(k0, k1, c0, c1):
    return threefry2x32_p.bind(k0, k1, c0, c1)


def _skey(k0, k1, j):
    r = _tf(k0, k1, jnp.zeros_like(k0), jnp.full_like(k0, j))
    return r[0], r[1]


def _rbits(k0, k1):
    r = _tf(k0, k1, jnp.zeros_like(k0), jnp.zeros_like(k0))
    return r[0] ^ r[1]


def _runif(k0, k1, lo, hi):
    bits = _rbits(k0, k1)
    fb = lax.shift_right_logical(bits, jnp.uint32(9)) | jnp.uint32(0x3F800000)
    f = lax.bitcast_convert_type(fb, jnp.float32) - np.float32(1.0)
    return lax.max(jnp.full_like(f, lo), f * (hi - lo) + lo)


def _rnormal(k0, k1):
    u = _runif(k0, k1, _NLO, np.float32(1.0))
    return _SQRT2 * lax.erf_inv(u)


def _posterior_body(nks_ref, sx_ref, sq_ref, mu_ref, cc_ref, rr_ref,
                    k0_ref, k1_ref, hm_ref, prec_ref):
    nks = nks_ref[...]
    sx = sx_ref[...]
    sq = sq_ref[...]
    shape = nks.shape
    mu = jnp.broadcast_to(mu_ref[...], shape)
    cc = jnp.broadcast_to(cc_ref[...], shape)
    rr = jnp.broadcast_to(rr_ref[...], shape)
    k0 = k0_ref[...]
    k1 = k1_ref[...]

    eff = nks + np.float32(1.0)
    hm = (mu + sx) / eff
    conc = cc + nks / np.float32(2.0)
    rt = rr + np.float32(0.5) * ((mu * mu - eff * (hm * hm)) + sq)

    alpha_orig = conc
    boost_mask = conc >= np.float32(1.0)
    alpha = jnp.where(boost_mask, conc, conc + np.float32(1.0))
    d = alpha - _F3
    c = _F3 / lax.sqrt(d)

    key0, key1 = _skey(k0, k1, 0)
    sub0, sub1 = _skey(k0, k1, 1)

    def rej_cond(x2, v3, u):
        c1 = u >= np.float32(1.0) - _SQUEEZE * (x2 * x2)
        c2 = lax.log(u) >= (x2 * np.float32(0.5)
                            + d * ((np.float32(1.0) - v3) + lax.log(v3)))
        return c1 & c2

    def obody(st):
        K0, K1, X, V, U = st
        m = rej_cond(X, V, U)
        nK0, nK1 = _skey(K0, K1, 0)
        xk0, xk1 = _skey(K0, K1, 1)
        uk0, uk1 = _skey(K0, K1, 2)

        def icond(ist):
            return jnp.any(ist[3] <= np.float32(0.0))

        def ibody(ist):
            ik0, ik1, x, v = ist
            im = v <= np.float32(0.0)
            jk0, jk1 = _skey(ik0, ik1, 0)
            sk0, sk1 = _skey(ik0, ik1, 1)
            xn = _rnormal(sk0, sk1)
            vn = np.float32(1.0) + xn * c
            return (jnp.where(im, jk0, ik0), jnp.where(im, jk1, ik1),
                    jnp.where(im, xn, x), jnp.where(im, vn, v))

        _, _, x, v = lax.while_loop(
            icond, ibody,
            (xk0, xk1, jnp.zeros_like(X), jnp.full_like(X, np.float32(-1.0))))
        nX = x * x
        nV = (v * v) * v
        nU = _runif(uk0, uk1, np.float32(0.0), np.float32(1.0))
        return (jnp.where(m, nK0, K0), jnp.where(m, nK1, K1),
                jnp.where(m, nX, X), jnp.where(m, nV, V), jnp.where(m, nU, U))

    def ocond(st):
        return jnp.any(rej_cond(st[2], st[3], st[4]))

    zf = jnp.zeros(shape, jnp.float32)
    _, _, _, Vf, _ = lax.while_loop(
        ocond, obody,
        (key0, key1, zf, zf + np.float32(1.0), zf + np.float32(2.0)))

    samples = np.float32(1.0) - _runif(sub0, sub1, np.float32(0.0), np.float32(1.0))
    boost = jnp.where(boost_mask, jnp.ones_like(samples),
                      lax.pow(samples, np.float32(1.0) / alpha_orig))
    gam = (d * Vf) * boost

    tau = gam / rt
    hm_ref[...] = hm
    prec_ref[...] = tau * eff


@jax.jit
def _cluster_stats(zs, x0, x1):
    mesh = plsc.VectorSubcoreMesh(core_axis_name="c", subcore_axis_name="s")
    params = pltpu.CompilerParams(needs_layout_passes=False)
    counts = pl.kernel(
        _counts_body,
        mesh=mesh,
        compiler_params=params,
        out_type=jax.ShapeDtypeStruct((BB, EW), jnp.float32),
        scratch_types=[
            pltpu.VMEM((CH,), jnp.int32),
            pltpu.VMEM((ACC,), jnp.float32),
            pltpu.VMEM((EW,), jnp.float32),
            pltpu.VMEM((CPB * EW,), jnp.float32),
            pltpu.VMEM_SHARED((NSUB * EW,), jnp.float32),
        ],
    )
    sums = pl.kernel(
        _sums_body,
        mesh=mesh,
        compiler_params=params,
        out_type=jax.ShapeDtypeStruct((BB, 2 * EW), jnp.float32),
        scratch_types=[
            pltpu.VMEM((CH,), jnp.int32),
            pltpu.VMEM((CH,), jnp.float32),
            pltpu.VMEM((CH,), jnp.float32),
            pltpu.VMEM((ACC,), jnp.float32),
            pltpu.VMEM((ACC,), jnp.float32),
            pltpu.VMEM((ACC,), jnp.float32),
            pltpu.VMEM((ACC,), jnp.float32),
            pltpu.VMEM((2 * EW,), jnp.float32),
            pltpu.VMEM((CPB * 2 * EW,), jnp.float32),
            pltpu.VMEM_SHARED((NSUB * 2 * EW,), jnp.float32),
        ],
    )
    return counts(zs), sums(zs, x0, x1)


def _posterior(nks2, sx, sq, muf, ccf, rrf, k0, k1):
    out = jax.ShapeDtypeStruct((BB, EW), jnp.float32)
    return pl.pallas_call(
        _posterior_body,
        out_shape=(out, out),
    )(nks2, sx, sq, muf, ccf, rrf, k0, k1)


def kernel(xs, zs, mu, concentration, rate):
    x0 = xs[..., 0]
    x1 = xs[..., 1]
    nks2, sums = _cluster_stats(zs.astype(jnp.int32), x0, x1)
    sx = sums[:, :EW]
    sq = sums[:, EW:]

    kd = jax.random.key_data(jax.random.split(jax.random.key(42), BB * EW))
    k0 = kd[:, 0].reshape(BB, EW)
    k1 = kd[:, 1].reshape(BB, EW)

    hm2, prec2 = _posterior(
        nks2, sx, sq,
        mu.reshape(1, EW), concentration.reshape(1, EW), rate.reshape(1, EW),
        k0, k1)

    hyper_means = hm2.reshape(BB, KC, 2)
    precisions = prec2.reshape(BB, KC, 2)
    nkey = jax.random.key(43)
    nrm = jax.random.normal(nkey, hyper_means.shape, dtype=xs.dtype)
    mu_sample = hyper_means + nrm * jnp.power(precisions, -0.5)
    return jnp.concatenate([hyper_means, precisions, mu_sample], axis=-1)


# speedup vs baseline: 1.1921x; 1.1921x over previous
import jax
import jax.numpy as jnp
import numpy as np
from jax import lax
from jax.experimental import pallas as pl
from jax.experimental.pallas import tpu as pltpu
from jax.experimental.pallas import tpu_sc as plsc
from jax._src.random.threefry2x32 import threefry2x32_p

KC = 64
LANES = 16
NCORES = 2
NSUB = 16
NW = NCORES * NSUB
BB = 4
NN = 65536
CPB = NW // BB
CH = NN // CPB
GROUPS = CH // LANES
ACC = KC * LANES
EW = 2 * KC


def _reduce_lanes(ref, colbase, off):
    def body(c, acc):
        return acc + plsc.load_gather(ref, [colbase + (off + c)])
    return lax.fori_loop(1, LANES, body, plsc.load_gather(ref, [colbase + off]))


def _combine_partials(s, b, part_v, tmp_v, shp, out_hbm, nvec):
    width = nvec * LANES
    pltpu.sync_copy(part_v, shp.at[pl.ds(s * width, width)])
    plsc.subcore_barrier()

    @pl.when(s % CPB == 0)
    def _():
        pltpu.sync_copy(shp.at[pl.ds(s * width, CPB * width)], tmp_v)

        def vbody(v, carry):
            def jbody(j, acc):
                return acc + tmp_v[pl.ds(j * width + v * LANES, LANES)]
            part_v[pl.ds(v * LANES, LANES)] = lax.fori_loop(
                1, CPB, jbody, tmp_v[pl.ds(v * LANES, LANES)])
            return carry

        lax.fori_loop(0, nvec, vbody, 0)
        pltpu.sync_copy(part_v.at[pl.ds(0, width)], out_hbm.at[b])


def _counts_body(zs_hbm, out_hbm, zs_v, cnt_v, part_v, tmp_v, shp):
    s = lax.axis_index("s")
    wid = lax.axis_index("c") * NSUB + s
    b = wid // CPB
    start = (wid % CPB) * CH

    pltpu.sync_copy(zs_hbm.at[b, pl.ds(start, CH)], zs_v)

    lane = lax.iota(jnp.int32, LANES)
    colbase = lane * LANES
    zeros = jnp.zeros((LANES,), jnp.float32)
    ones = jnp.ones((LANES,), jnp.float32)

    @plsc.parallel_loop(0, ACC // LANES, unroll=4)
    def _(i):
        cnt_v[pl.ds(i * LANES, LANES)] = zeros

    @plsc.parallel_loop(0, GROUPS, unroll=4)
    def _(i):
        z = zs_v[pl.ds(i * LANES, LANES)]
        plsc.addupdate_scatter(cnt_v, [z * LANES + lane], ones)

    for ch in range(KC // LANES):
        v = _reduce_lanes(cnt_v, colbase, ch * LANES * LANES)
        idx = lane * 2 + (ch * 2 * LANES)
        plsc.store_scatter(part_v, [idx], v)
        plsc.store_scatter(part_v, [idx + 1], v)

    _combine_partials(s, b, part_v, tmp_v, shp, out_hbm, EW // LANES)


def _sums_body(zs_hbm, x0_hbm, x1_hbm, out_hbm,
               zs_v, x0_v, x1_v, sx0_v, sx1_v, sq0_v, sq1_v, part_v, tmp_v, shp):
    s = lax.axis_index("s")
    wid = lax.axis_index("c") * NSUB + s
    b = wid // CPB
    start = (wid % CPB) * CH

    pltpu.sync_copy(zs_hbm.at[b, pl.ds(start, CH)], zs_v)
    pltpu.sync_copy(x0_hbm.at[b, pl.ds(start, CH)], x0_v)
    pltpu.sync_copy(x1_hbm.at[b, pl.ds(start, CH)], x1_v)

    lane = lax.iota(jnp.int32, LANES)
    colbase = lane * LANES
    zeros = jnp.zeros((LANES,), jnp.float32)

    @plsc.parallel_loop(0, ACC // LANES, unroll=4)
    def _(i):
        sl = pl.ds(i * LANES, LANES)
        sx0_v[sl] = zeros
        sx1_v[sl] = zeros
        sq0_v[sl] = zeros
        sq1_v[sl] = zeros

    @plsc.parallel_loop(0, GROUPS, unroll=2)
    def _(i):
        sl = pl.ds(i * LANES, LANES)
        z = zs_v[sl]
        x0 = x0_v[sl]
        x1 = x1_v[sl]
        idx = z * LANES + lane
        plsc.addupdate_scatter(sx0_v, [idx], x0)
        plsc.addupdate_scatter(sx1_v, [idx], x1)
        plsc.addupdate_scatter(sq0_v, [idx], x0 * x0)
        plsc.addupdate_scatter(sq1_v, [idx], x1 * x1)

    for ch in range(KC // LANES):
        off = ch * LANES * LANES
        base = ch * 2 * LANES
        idx = lane * 2 + base
        plsc.store_scatter(part_v, [idx], _reduce_lanes(sx0_v, colbase, off))
        plsc.store_scatter(part_v, [idx + 1], _reduce_lanes(sx1_v, colbase, off))
        plsc.store_scatter(part_v, [idx + EW], _reduce_lanes(sq0_v, colbase, off))
        plsc.store_scatter(part_v, [idx + EW + 1], _reduce_lanes(sq1_v, colbase, off))

    _combine_partials(s, b, part_v, tmp_v, shp, out_hbm, 2 * EW // LANES)


_F3 = np.float32(1.0 / 3.0)
_SQUEEZE = np.float32(0.0331)
_NLO = np.nextafter(np.float32(-1.0), np.float32(0.0), dtype=np.float32)
_SQRT2 = np.float32(np.sqrt(2.0))


def _tf(k0, k1, c0, c1):
    return threefry2x32_p.bind(k0, k1, c0, c1)


def _skey(k0, k1, j):
    r = _tf(k0, k1, jnp.zeros_like(k0), jnp.full_like(k0, j))
    return r[0], r[1]


def _rbits(k0, k1):
    r = _tf(k0, k1, jnp.zeros_like(k0), jnp.zeros_like(k0))
    return r[0] ^ r[1]


def _runif(k0, k1, lo, hi):
    bits = _rbits(k0, k1)
    fb = lax.shift_right_logical(bits, jnp.uint32(9)) | jnp.uint32(0x3F800000)
    f = lax.bitcast_convert_type(fb, jnp.float32) - np.float32(1.0)
    return lax.max(jnp.full_like(f, lo), f * (hi - lo) + lo)


def _rnormal(k0, k1):
    u = _runif(k0, k1, _NLO, np.float32(1.0))
    return _SQRT2 * lax.erf_inv(u)


def _gamma_body(nks_ref, cc_ref, k0_ref, k1_ref, gam_ref):
    nks = nks_ref[...]
    shape = nks.shape
    cc = jnp.broadcast_to(cc_ref[...], shape)
    k0 = k0_ref[...]
    k1 = k1_ref[...]

    conc = cc + nks / np.float32(2.0)

    alpha_orig = conc
    boost_mask = conc >= np.float32(1.0)
    alpha = jnp.where(boost_mask, conc, conc + np.float32(1.0))
    d = alpha - _F3
    c = _F3 / lax.sqrt(d)

    key0, key1 = _skey(k0, k1, 0)
    sub0, sub1 = _skey(k0, k1, 1)

    def rej_cond(x2, v3, u):
        c1 = u >= np.float32(1.0) - _SQUEEZE * (x2 * x2)
        c2 = lax.log(u) >= (x2 * np.float32(0.5)
                            + d * ((np.float32(1.0) - v3) + lax.log(v3)))
        return c1 & c2

    def obody(st):
        K0, K1, X, V, U = st
        m = rej_cond(X, V, U)
        nK0, nK1 = _skey(K0, K1, 0)
        xk0, xk1 = _skey(K0, K1, 1)
        uk0, uk1 = _skey(K0, K1, 2)

        def icond(ist):
            return jnp.any(ist[3] <= np.float32(0.0))

        def ibody(ist):
            ik0, ik1, x, v = ist
            im = v <= np.float32(0.0)
            jk0, jk1 = _skey(ik0, ik1, 0)
            sk0, sk1 = _skey(ik0, ik1, 1)
            xn = _rnormal(sk0, sk1)
            vn = np.float32(1.0) + xn * c
            return (jnp.where(im, jk0, ik0), jnp.where(im, jk1, ik1),
                    jnp.where(im, xn, x), jnp.where(im, vn, v))

        _, _, x, v = lax.while_loop(
            icond, ibody,
            (xk0, xk1, jnp.zeros_like(X), jnp.full_like(X, np.float32(-1.0))))
        nX = x * x
        nV = (v * v) * v
        nU = _runif(uk0, uk1, np.float32(0.0), np.float32(1.0))
        return (jnp.where(m, nK0, K0), jnp.where(m, nK1, K1),
                jnp.where(m, nX, X), jnp.where(m, nV, V), jnp.where(m, nU, U))

    def ocond(st):
        return jnp.any(rej_cond(st[2], st[3], st[4]))

    zf = jnp.zeros(shape, jnp.float32)
    _, _, _, Vf, _ = lax.while_loop(
        ocond, obody,
        (key0, key1, zf, zf + np.float32(1.0), zf + np.float32(2.0)))

    samples = np.float32(1.0) - _runif(sub0, sub1, np.float32(0.0), np.float32(1.0))
    boost = jnp.where(boost_mask, jnp.ones_like(samples),
                      lax.pow(samples, np.float32(1.0) / alpha_orig))
    gam_ref[...] = (d * Vf) * boost


@jax.jit
def _cluster_stats(zs, x0, x1):
    mesh = plsc.VectorSubcoreMesh(core_axis_name="c", subcore_axis_name="s")
    params = pltpu.CompilerParams(needs_layout_passes=False)
    counts = pl.kernel(
        _counts_body,
        mesh=mesh,
        compiler_params=params,
        out_type=jax.ShapeDtypeStruct((BB, EW), jnp.float32),
        scratch_types=[
            pltpu.VMEM((CH,), jnp.int32),
            pltpu.VMEM((ACC,), jnp.float32),
            pltpu.VMEM((EW,), jnp.float32),
            pltpu.VMEM((CPB * EW,), jnp.float32),
            pltpu.VMEM_SHARED((NSUB * EW,), jnp.float32),
        ],
    )
    sums = pl.kernel(
        _sums_body,
        mesh=mesh,
        compiler_params=params,
        out_type=jax.ShapeDtypeStruct((BB, 2 * EW), jnp.float32),
        scratch_types=[
            pltpu.VMEM((CH,), jnp.int32),
            pltpu.VMEM((CH,), jnp.float32),
            pltpu.VMEM((CH,), jnp.float32),
            pltpu.VMEM((ACC,), jnp.float32),
            pltpu.VMEM((ACC,), jnp.float32),
            pltpu.VMEM((ACC,), jnp.float32),
            pltpu.VMEM((ACC,), jnp.float32),
            pltpu.VMEM((2 * EW,), jnp.float32),
            pltpu.VMEM((CPB * 2 * EW,), jnp.float32),
            pltpu.VMEM_SHARED((NSUB * 2 * EW,), jnp.float32),
        ],
    )
    return counts(zs), sums(zs, x0, x1)


def _gamma_tc(nks2, ccf, k0, k1):
    out = jax.ShapeDtypeStruct((BB, EW), jnp.float32)
    return pl.pallas_call(
        _gamma_body,
        out_shape=out,
    )(nks2, ccf, k0, k1)


def kernel(xs, zs, mu, concentration, rate):
    x0 = xs[..., 0]
    x1 = xs[..., 1]
    nks2, sums = _cluster_stats(zs.astype(jnp.int32), x0, x1)
    sx = sums[:, :EW]
    sq = sums[:, EW:]

    kd = jax.random.key_data(jax.random.split(jax.random.key(42), BB * EW))
    k0 = kd[:, 0].reshape(BB, EW)
    k1 = kd[:, 1].reshape(BB, EW)

    gam = _gamma_tc(nks2, concentration.reshape(1, EW), k0, k1)

    muf = mu.reshape(1, EW)
    eff = nks2 + 1.0
    hm2 = (muf + sx) / eff
    rt = rate.reshape(1, EW) + 0.5 * ((muf * muf - eff * (hm2 * hm2)) + sq)
    prec2 = (gam / rt) * eff

    hyper_means = hm2.reshape(BB, KC, 2)
    precisions = prec2.reshape(BB, KC, 2)
    nkey = jax.random.key(43)
    nrm = jax.random.normal(nkey, hyper_means.shape, dtype=xs.dtype)
    mu_sample = hyper_means + nrm * jnp.power(precisions, -0.5)
    return jnp.concatenate([hyper_means, precisions, mu_sample], axis=-1)


# speedup vs baseline: 1.1928x; 1.0005x over previous
import jax
import jax.numpy as jnp
import numpy as np
from jax import lax
from jax.experimental import pallas as pl
from jax.experimental.pallas import tpu as pltpu
from jax.experimental.pallas import tpu_sc as plsc
from jax._src.random.threefry2x32 import threefry2x32_p

KC = 64
LANES = 16
NCORES = 2
NSUB = 16
NW = NCORES * NSUB
BB = 4
NN = 65536
CPB = NW // BB
CH = NN // CPB
GROUPS = CH // LANES
ACC = KC * LANES
EW = 2 * KC


def _reduce_lanes(ref, colbase, off):
    def body(c, acc):
        return acc + plsc.load_gather(ref, [colbase + (off + c)])
    return lax.fori_loop(1, LANES, body, plsc.load_gather(ref, [colbase + off]))


def _combine_partials(s, b, part_v, tmp_v, shp, out_hbm, nvec):
    width = nvec * LANES
    pltpu.sync_copy(part_v, shp.at[pl.ds(s * width, width)])
    plsc.subcore_barrier()

    @pl.when(s % CPB == 0)
    def _():
        pltpu.sync_copy(shp.at[pl.ds(s * width, CPB * width)], tmp_v)

        def vbody(v, carry):
            def jbody(j, acc):
                return acc + tmp_v[pl.ds(j * width + v * LANES, LANES)]
            part_v[pl.ds(v * LANES, LANES)] = lax.fori_loop(
                1, CPB, jbody, tmp_v[pl.ds(v * LANES, LANES)])
            return carry

        lax.fori_loop(0, nvec, vbody, 0)
        pltpu.sync_copy(part_v.at[pl.ds(0, width)], out_hbm.at[b])


def _counts_body(zs_hbm, out_hbm, zs_v, cnt_v, part_v, tmp_v, shp):
    s = lax.axis_index("s")
    wid = lax.axis_index("c") * NSUB + s
    b = wid // CPB
    start = (wid % CPB) * CH

    pltpu.sync_copy(zs_hbm.at[b, pl.ds(start, CH)], zs_v)

    lane = lax.iota(jnp.int32, LANES)
    colbase = lane * LANES
    zeros = jnp.zeros((LANES,), jnp.float32)
    ones = jnp.ones((LANES,), jnp.float32)

    @plsc.parallel_loop(0, ACC // LANES, unroll=4)
    def _(i):
        cnt_v[pl.ds(i * LANES, LANES)] = zeros

    @plsc.parallel_loop(0, GROUPS, unroll=4)
    def _(i):
        z = zs_v[pl.ds(i * LANES, LANES)]
        plsc.addupdate_scatter(cnt_v, [z * LANES + lane], ones)

    for ch in range(KC // LANES):
        v = _reduce_lanes(cnt_v, colbase, ch * LANES * LANES)
        idx = lane * 2 + (ch * 2 * LANES)
        plsc.store_scatter(part_v, [idx], v)
        plsc.store_scatter(part_v, [idx + 1], v)

    _combine_partials(s, b, part_v, tmp_v, shp, out_hbm, EW // LANES)


def _sums_body(zs_hbm, x0_hbm, x1_hbm, out_hbm,
               zs_v, x0_v, x1_v, sx0_v, sx1_v, sq0_v, sq1_v, part_v, tmp_v, shp):
    s = lax.axis_index("s")
    wid = lax.axis_index("c") * NSUB + s
    b = wid // CPB
    start = (wid % CPB) * CH

    pltpu.sync_copy(zs_hbm.at[b, pl.ds(start, CH)], zs_v)
    pltpu.sync_copy(x0_hbm.at[b, pl.ds(start, CH)], x0_v)
    pltpu.sync_copy(x1_hbm.at[b, pl.ds(start, CH)], x1_v)

    lane = lax.iota(jnp.int32, LANES)
    colbase = lane * LANES
    zeros = jnp.zeros((LANES,), jnp.float32)

    @plsc.parallel_loop(0, ACC // LANES, unroll=4)
    def _(i):
        sl = pl.ds(i * LANES, LANES)
        sx0_v[sl] = zeros
        sx1_v[sl] = zeros
        sq0_v[sl] = zeros
        sq1_v[sl] = zeros

    @plsc.parallel_loop(0, GROUPS, unroll=4)
    def _(i):
        sl = pl.ds(i * LANES, LANES)
        z = zs_v[sl]
        x0 = x0_v[sl]
        x1 = x1_v[sl]
        idx = z * LANES + lane
        plsc.addupdate_scatter(sx0_v, [idx], x0)
        plsc.addupdate_scatter(sx1_v, [idx], x1)
        plsc.addupdate_scatter(sq0_v, [idx], x0 * x0)
        plsc.addupdate_scatter(sq1_v, [idx], x1 * x1)

    for ch in range(KC // LANES):
        off = ch * LANES * LANES
        base = ch * 2 * LANES
        idx = lane * 2 + base
        plsc.store_scatter(part_v, [idx], _reduce_lanes(sx0_v, colbase, off))
        plsc.store_scatter(part_v, [idx + 1], _reduce_lanes(sx1_v, colbase, off))
        plsc.store_scatter(part_v, [idx + EW], _reduce_lanes(sq0_v, colbase, off))
        plsc.store_scatter(part_v, [idx + EW + 1], _reduce_lanes(sq1_v, colbase, off))

    _combine_partials(s, b, part_v, tmp_v, shp, out_hbm, 2 * EW // LANES)


_F3 = np.float32(1.0 / 3.0)
_SQUEEZE = np.float32(0.0331)
_NLO = np.nextafter(np.float32(-1.0), np.float32(0.0), dtype=np.float32)
_SQRT2 = np.float32(np.sqrt(2.0))


def _tf(k0, k1, c0, c1):
    return threefry2x32_p.bind(k0, k1, c0, c1)


def _skey(k0, k1, j):
    r = _tf(k0, k1, jnp.zeros_like(k0), jnp.full_like(k0, j))
    return r[0], r[1]


def _rbits(k0, k1):
    r = _tf(k0, k1, jnp.zeros_like(k0), jnp.zeros_like(k0))
    return r[0] ^ r[1]


def _runif(k0, k1, lo, hi):
    bits = _rbits(k0, k1)
    fb = lax.shift_right_logical(bits, jnp.uint32(9)) | jnp.uint32(0x3F800000)
    f = lax.bitcast_convert_type(fb, jnp.float32) - np.float32(1.0)
    return lax.max(jnp.full_like(f, lo), f * (hi - lo) + lo)


def _rnormal(k0, k1):
    u = _runif(k0, k1, _NLO, np.float32(1.0))
    return _SQRT2 * lax.erf_inv(u)


def _gamma_body(nks_ref, cc_ref, k0_ref, k1_ref, gam_ref):
    nks = nks_ref[...]
    shape = nks.shape
    cc = jnp.broadcast_to(cc_ref[...], shape)
    k0 = k0_ref[...]
    k1 = k1_ref[...]

    conc = cc + nks / np.float32(2.0)

    alpha_orig = conc
    boost_mask = conc >= np.float32(1.0)
    alpha = jnp.where(boost_mask, conc, conc + np.float32(1.0))
    d = alpha - _F3
    c = _F3 / lax.sqrt(d)

    key0, key1 = _skey(k0, k1, 0)
    sub0, sub1 = _skey(k0, k1, 1)

    def rej_cond(x2, v3, u):
        c1 = u >= np.float32(1.0) - _SQUEEZE * (x2 * x2)
        c2 = lax.log(u) >= (x2 * np.float32(0.5)
                            + d * ((np.float32(1.0) - v3) + lax.log(v3)))
        return c1 & c2

    def obody(st):
        K0, K1, X, V, U = st
        m = rej_cond(X, V, U)
        nK0, nK1 = _skey(K0, K1, 0)
        xk0, xk1 = _skey(K0, K1, 1)
        uk0, uk1 = _skey(K0, K1, 2)

        def icond(ist):
            return jnp.any(ist[3] <= np.float32(0.0))

        def ibody(ist):
            ik0, ik1, x, v = ist
            im = v <= np.float32(0.0)
            jk0, jk1 = _skey(ik0, ik1, 0)
            sk0, sk1 = _skey(ik0, ik1, 1)
            xn = _rnormal(sk0, sk1)
            vn = np.float32(1.0) + xn * c
            return (jnp.where(im, jk0, ik0), jnp.where(im, jk1, ik1),
                    jnp.where(im, xn, x), jnp.where(im, vn, v))

        _, _, x, v = lax.while_loop(
            icond, ibody,
            (xk0, xk1, jnp.zeros_like(X), jnp.full_like(X, np.float32(-1.0))))
        nX = x * x
        nV = (v * v) * v
        nU = _runif(uk0, uk1, np.float32(0.0), np.float32(1.0))
        return (jnp.where(m, nK0, K0), jnp.where(m, nK1, K1),
                jnp.where(m, nX, X), jnp.where(m, nV, V), jnp.where(m, nU, U))

    def ocond(st):
        return jnp.any(rej_cond(st[2], st[3], st[4]))

    zf = jnp.zeros(shape, jnp.float32)
    _, _, _, Vf, _ = lax.while_loop(
        ocond, obody,
        (key0, key1, zf, zf + np.float32(1.0), zf + np.float32(2.0)))

    samples = np.float32(1.0) - _runif(sub0, sub1, np.float32(0.0), np.float32(1.0))
    boost = jnp.where(boost_mask, jnp.ones_like(samples),
                      lax.pow(samples, np.float32(1.0) / alpha_orig))
    gam_ref[...] = (d * Vf) * boost


@jax.jit
def _cluster_stats(zs, x0, x1):
    mesh = plsc.VectorSubcoreMesh(core_axis_name="c", subcore_axis_name="s")
    params = pltpu.CompilerParams(needs_layout_passes=False)
    counts = pl.kernel(
        _counts_body,
        mesh=mesh,
        compiler_params=params,
        out_type=jax.ShapeDtypeStruct((BB, EW), jnp.float32),
        scratch_types=[
            pltpu.VMEM((CH,), jnp.int32),
            pltpu.VMEM((ACC,), jnp.float32),
            pltpu.VMEM((EW,), jnp.float32),
            pltpu.VMEM((CPB * EW,), jnp.float32),
            pltpu.VMEM_SHARED((NSUB * EW,), jnp.float32),
        ],
    )
    sums = pl.kernel(
        _sums_body,
        mesh=mesh,
        compiler_params=params,
        out_type=jax.ShapeDtypeStruct((BB, 2 * EW), jnp.float32),
        scratch_types=[
            pltpu.VMEM((CH,), jnp.int32),
            pltpu.VMEM((CH,), jnp.float32),
            pltpu.VMEM((CH,), jnp.float32),
            pltpu.VMEM((ACC,), jnp.float32),
            pltpu.VMEM((ACC,), jnp.float32),
            pltpu.VMEM((ACC,), jnp.float32),
            pltpu.VMEM((ACC,), jnp.float32),
            pltpu.VMEM((2 * EW,), jnp.float32),
            pltpu.VMEM((CPB * 2 * EW,), jnp.float32),
            pltpu.VMEM_SHARED((NSUB * 2 * EW,), jnp.float32),
        ],
    )
    return counts(zs), sums(zs, x0, x1)


def _gamma_tc(nks2, ccf, k0, k1):
    out = jax.ShapeDtypeStruct((BB, EW), jnp.float32)
    return pl.pallas_call(
        _gamma_body,
        out_shape=out,
    )(nks2, ccf, k0, k1)


def kernel(xs, zs, mu, concentration, rate):
    x0 = xs[..., 0]
    x1 = xs[..., 1]
    nks2, sums = _cluster_stats(zs.astype(jnp.int32), x0, x1)
    sx = sums[:, :EW]
    sq = sums[:, EW:]

    kd = jax.random.key_data(jax.random.split(jax.random.key(42), BB * EW))
    k0 = kd[:, 0].reshape(BB, EW)
    k1 = kd[:, 1].reshape(BB, EW)

    gam = _gamma_tc(nks2, concentration.reshape(1, EW), k0, k1)

    muf = mu.reshape(1, EW)
    eff = nks2 + 1.0
    hm2 = (muf + sx) / eff
    rt = rate.reshape(1, EW) + 0.5 * ((muf * muf - eff * (hm2 * hm2)) + sq)
    prec2 = (gam / rt) * eff

    hyper_means = hm2.reshape(BB, KC, 2)
    precisions = prec2.reshape(BB, KC, 2)
    nkey = jax.random.key(43)
    nrm = jax.random.normal(nkey, hyper_means.shape, dtype=xs.dtype)
    mu_sample = hyper_means + nrm * jnp.power(precisions, -0.5)
    return jnp.concatenate([hyper_means, precisions, mu_sample], axis=-1)
